# baseline (speedup 1.0000x reference)
import jax
import jax.numpy as jnp
from jax import lax
from jax.experimental import pallas as pl
from jax.experimental.pallas import tpu as pltpu
from jax.experimental.pallas import tpu_sc as plsc

_VOCAB = 1000000
_EMBED = 64
_S = 4096
_T = 200
_B = _S * _T

_NC = 2
_NS = 16
_NW = _NC * _NS

_CHUNK = 128
_GPB = 2
_B_PER_W = _B // _NW
_CHUNKS_PER_W = _B_PER_W // _CHUNK
_SUPERS_PER_W = _CHUNKS_PER_W // _GPB
_SB = _S // _CHUNK

_VB = 32768


def _detile_body(x_ref, o_ref):
  o_ref[:, 0:_EMBED] = x_ref[...].T


def _table_wide(w_t):
  grid = (_VOCAB + _VB - 1) // _VB
  return pl.pallas_call(
      _detile_body,
      grid=(grid,),
      in_specs=[pl.BlockSpec((_EMBED, _VB), lambda i: (0, i))],
      out_specs=pl.BlockSpec((_VB, 2 * _EMBED), lambda i: (i, 0)),
      out_shape=jax.ShapeDtypeStruct((_VOCAB, 2 * _EMBED), jnp.float32),
  )(w_t)


def _body(table_hbm, idx_hbm, out_hbm,
          idx_v, rows0, rows1, rows2, st0, st1, st2,
          sem_g0, sem_g1, sem_g2, sem_o0, sem_o1, sem_o2):
  wid = lax.axis_index("s") * _NC + lax.axis_index("c")
  base_c = wid * _CHUNKS_PER_W

  pltpu.sync_copy(idx_hbm.at[wid], idx_v)

  rows = (rows0, rows1, rows2)
  st = (st0, st1, st2)
  sems = (sem_g0, sem_g1, sem_g2)
  sem_o = (sem_o0, sem_o1, sem_o2)

  def fire(b, m):
    pltpu.async_copy(table_hbm.at[idx_v.at[m]], rows[b], sems[b])

  def drain(b):
    pltpu.make_async_copy(
        table_hbm.at[pl.ds(0, _CHUNK)], rows[b], sems[b]).wait()

  def store(b, m):
    c = base_c + m
    t = c // _SB
    s0 = (c % _SB) * _CHUNK
    pltpu.make_async_copy(
        out_hbm.at[0, pl.ds(0, _CHUNK)], st[b], sem_o[b]).wait()

    def row_step(l, carry):
      for k in range(_EMBED // 16):
        st[b][l, pl.ds(k * 16, 16)] = rows[b][l, pl.ds(k * 16, 16)]
      return carry

    lax.fori_loop(0, _CHUNK, row_step, 0, unroll=16)
    pltpu.async_copy(st[b], out_hbm.at[t, pl.ds(s0, _CHUNK)], sem_o[b])

  t0 = base_c // _SB
  sb0 = (base_c % _SB) * _CHUNK
  for b in range(3):
    pltpu.async_copy(st[b], out_hbm.at[t0, pl.ds(sb0, _CHUNK)], sem_o[b])

  for b in range(3):
    fire(b, b)

  def step(m3, carry):
    for b in range(3):
      m = m3 * 3 + b
      drain(b)
      store(b, m)
      fire(b, m + 3)
    return carry

  lax.fori_loop(0, _CHUNKS_PER_W // 3 - 1, step, 0)

  for m in range(3 * (_CHUNKS_PER_W // 3 - 1), _CHUNKS_PER_W):
    b = m % 3
    drain(b)
    store(b, m)
    if m + 3 < _CHUNKS_PER_W:
      fire(b, m + 3)
  for b in range(3):
    pltpu.make_async_copy(
        out_hbm.at[0, pl.ds(0, _CHUNK)], st[b], sem_o[b]).wait()


@jax.jit
def kernel(sentence, W_word):
  idx = sentence.T.astype(jnp.int32).reshape(_NW, _CHUNKS_PER_W, _CHUNK)
  table = _table_wide(W_word.T)
  mesh = plsc.VectorSubcoreMesh(core_axis_name="c", subcore_axis_name="s")
  out = pl.kernel(
      _body,
      out_type=jax.ShapeDtypeStruct((_T, _S, _EMBED), jnp.float32),
      mesh=mesh,
      scratch_types=(
          [pltpu.VMEM((_CHUNKS_PER_W, _CHUNK), jnp.int32)]
          + [pltpu.VMEM((_CHUNK, 2 * _EMBED), jnp.float32)] * 3
          + [pltpu.VMEM((_CHUNK, _EMBED), jnp.float32)] * 3
          + [pltpu.SemaphoreType.DMA] * 6
      ),
      compiler_params=pltpu.CompilerParams(use_tc_tiling_on_sc=True),
  )(table, idx)
  return out.swapaxes(0, 1)


# speedup vs baseline: 1.0880x; 1.0880x over previous
import jax
import jax.numpy as jnp
from jax import lax
from jax.experimental import pallas as pl
from jax.experimental.pallas import tpu as pltpu
from jax.experimental.pallas import tpu_sc as plsc

_VOCAB = 1000000
_EMBED = 64
_S = 4096
_T = 200
_B = _S * _T

_NC = 2
_NS = 16
_NW = _NC * _NS

_CHUNK = 128
_GPB = 2
_B_PER_W = _B // _NW
_CHUNKS_PER_W = _B_PER_W // _CHUNK
_SUPERS_PER_W = _CHUNKS_PER_W // _GPB
_SB = _S // _CHUNK

_VB = 32768


def _detile_body(x_ref, o_ref):
  eye = jnp.eye(_EMBED, dtype=jnp.float32)
  o_ref[:, 0:_EMBED] = jax.lax.dot_general(
      x_ref[...], eye, (((0,), (0,)), ((), ())),
      preferred_element_type=jnp.float32)


def _table_wide(w_t):
  grid = (_VOCAB + _VB - 1) // _VB
  return pl.pallas_call(
      _detile_body,
      grid=(grid,),
      in_specs=[pl.BlockSpec((_EMBED, _VB), lambda i: (0, i))],
      out_specs=pl.BlockSpec((_VB, 2 * _EMBED), lambda i: (i, 0)),
      out_shape=jax.ShapeDtypeStruct((_VOCAB, 2 * _EMBED), jnp.float32),
  )(w_t)


def _body(table_hbm, idx_hbm, out_hbm,
          idx_v, rows0, rows1, st0, st1, sem_g0, sem_g1, sem_o0, sem_o1):
  wid = lax.axis_index("s") * _NC + lax.axis_index("c")
  base_c = wid * _CHUNKS_PER_W

  pltpu.sync_copy(idx_hbm.at[wid], idx_v)

  rows = (rows0, rows1)
  st = (st0, st1)
  sems = (sem_g0, sem_g1)
  sem_o = (sem_o0, sem_o1)

  def fire(b, s):
    for j in range(_GPB):
      pltpu.async_copy(
          table_hbm.at[idx_v.at[s * _GPB + j]],
          rows[b].at[j],
          sems[b],
      )

  def drain(b):
    for j in range(_GPB):
      pltpu.make_async_copy(
          table_hbm.at[pl.ds(0, _CHUNK)], rows[b].at[j], sems[b]).wait()

  def store(b, s):
    for j in range(_GPB):
      c = base_c + s * _GPB + j
      t = c // _SB
      s0 = (c % _SB) * _CHUNK
      pltpu.make_async_copy(
          out_hbm.at[0, pl.ds(0, _CHUNK)], st[j], sem_o[j]).wait()

      def row_step(l, carry):
        for k in range(_EMBED // 16):
          st[j][l, pl.ds(k * 16, 16)] = rows[b][j, l, pl.ds(k * 16, 16)]
        return carry

      lax.fori_loop(0, _CHUNK, row_step, 0, unroll=16)
      pltpu.async_copy(st[j], out_hbm.at[t, pl.ds(s0, _CHUNK)], sem_o[j])

  t0 = base_c // _SB
  sb0 = (base_c % _SB) * _CHUNK
  pltpu.async_copy(st0, out_hbm.at[t0, pl.ds(sb0, _CHUNK)], sem_o0)
  pltpu.async_copy(st1, out_hbm.at[t0, pl.ds(sb0, _CHUNK)], sem_o1)

  fire(0, 0)
  fire(1, 1)

  def step(s2, carry):
    for b in range(2):
      s = s2 * 2 + b
      drain(b)
      store(b, s)
      fire(b, s + 2)
    return carry

  lax.fori_loop(0, _SUPERS_PER_W // 2 - 1, step, 0)

  for b in range(2):
    s = _SUPERS_PER_W - 2 + b
    drain(b)
    store(b, s)
  for j in range(2):
    pltpu.make_async_copy(
        out_hbm.at[0, pl.ds(0, _CHUNK)], st[j], sem_o[j]).wait()


@jax.jit
def kernel(sentence, W_word):
  idx = sentence.T.astype(jnp.int32).reshape(_NW, _CHUNKS_PER_W, _CHUNK)
  table = _table_wide(W_word.T)
  mesh = plsc.VectorSubcoreMesh(core_axis_name="c", subcore_axis_name="s")
  out = pl.kernel(
      _body,
      out_type=jax.ShapeDtypeStruct((_T, _S, _EMBED), jnp.float32),
      mesh=mesh,
      scratch_types=[
          pltpu.VMEM((_CHUNKS_PER_W, _CHUNK), jnp.int32),
          pltpu.VMEM((_GPB, _CHUNK, 2 * _EMBED), jnp.float32),
          pltpu.VMEM((_GPB, _CHUNK, 2 * _EMBED), jnp.float32),
          pltpu.VMEM((_CHUNK, _EMBED), jnp.float32),
          pltpu.VMEM((_CHUNK, _EMBED), jnp.float32),
          pltpu.SemaphoreType.DMA,
          pltpu.SemaphoreType.DMA,
          pltpu.SemaphoreType.DMA,
          pltpu.SemaphoreType.DMA,
      ],
      compiler_params=pltpu.CompilerParams(use_tc_tiling_on_sc=True),
  )(table, idx)
  return out.swapaxes(0, 1)
